# baseline (speedup 1.0000x reference)
import functools

import jax
import jax.numpy as jnp
from jax import lax
from jax.experimental import pallas as pl
from jax.experimental.pallas import tpu as pltpu


def _log_shift_cumsum_rows(x, row_iota, rbits):
    t = x
    for m in range(rbits):
        s = 1 << m
        t = t + jnp.where(row_iota >= s, jnp.roll(t, s, axis=0), 0.0)
    return t


def _lane_exclusive_prefix(v, col_iota1, lbits):
    t = v
    for m in range(lbits):
        s = 1 << m
        t = t + jnp.where(col_iota1 >= s, jnp.roll(t, s, axis=1), 0.0)
    return t - v


def _partner_branches(R, L, nbits, col):
    rbits = R.bit_length() - 1
    branches = []
    for j in range(nbits):
        d = 1 << j
        if d < R:
            def row_swap(x, d=d):
                x4 = x.reshape(R // (2 * d), 2, d, L)
                sw = jnp.concatenate([x4[:, 1:2], x4[:, 0:1]], axis=1)
                return sw.reshape(R, L)
            branches.append(row_swap)
        else:
            ld = d >> rbits

            def lane_swap(x, ld=ld):
                up = jnp.roll(x, -ld, axis=1)
                down = jnp.roll(x, ld, axis=1)
                return jnp.where((col & ld) == 0, up, down)
            branches.append(lane_swap)
    return branches


def _lovasz_body(inp_ref, tgt_ref, out_ref, acc_ref, *, R, L, C):
    P = R * L
    nbits = P.bit_length() - 1
    rbits = R.bit_length() - 1
    lbits = L.bit_length() - 1
    c = pl.program_id(0)

    @pl.when(c == 0)
    def _init():
        acc_ref[0] = 0.0
        acc_ref[1] = 0.0

    x = inp_ref[0]
    t = tgt_ref[...]
    fgb = (t == c).astype(jnp.int32)
    fg = fgb.astype(jnp.float32)
    p = jnp.exp(x)
    e = jnp.abs(fg - p)
    key = (lax.bitcast_convert_type(e, jnp.int32) & jnp.int32(-2)) | fgb

    row = lax.broadcasted_iota(jnp.int32, (R, L), 0)
    col = lax.broadcasted_iota(jnp.int32, (R, L), 1)
    l = col * R + row

    branches = _partner_branches(R, L, nbits, col)

    def layer(key, k, j):
        partner = lax.switch(j, branches, key)
        bits = jnp.right_shift(l, k) ^ jnp.right_shift(l, j)
        keep_max = (bits & 1) == 0
        mn = jnp.minimum(key, partner)
        mx = jnp.maximum(key, partner)
        return jnp.where(keep_max, mx, mn)

    def stage(k, key):
        def inner(s, key):
            return layer(key, k, k - 1 - s)
        return lax.fori_loop(0, k, inner, key)

    key = lax.fori_loop(1, nbits + 1, stage, key)

    f = (key & 1).astype(jnp.float32)
    es = lax.bitcast_convert_type(key & jnp.int32(-2), jnp.float32)

    S_col = _log_shift_cumsum_rows(f, row, rbits)
    colsum = S_col[R - 1 : R, :]
    col1 = lax.broadcasted_iota(jnp.int32, (1, L), 1)
    off = _lane_exclusive_prefix(colsum, col1, lbits)
    S = S_col + off
    G = jnp.sum(f)

    kf = (l + 1).astype(jnp.float32)
    union = G + kf - S
    jac = 1.0 - (G - S) / union

    jac_prev = jnp.roll(jac, 1, axis=0)
    lastrow = jac[R - 1 : R, :]
    lsr = jnp.where(col1 >= 1, jnp.roll(lastrow, 1, axis=1), 0.0)
    jac_prev = jnp.where(row == 0, lsr, jac_prev)
    grad = jac - jac_prev

    loss_c = jnp.sum(es * grad)
    present = (G > 0.0).astype(jnp.float32)

    acc_ref[0] = acc_ref[0] + loss_c * present
    acc_ref[1] = acc_ref[1] + present
    cnt = acc_ref[1]
    res = jnp.where(cnt > 0.0, acc_ref[0] / cnt, 0.0)
    out_ref[...] = jnp.broadcast_to(res, (1, 1))


def kernel(inputs, targets):
    P, C = inputs.shape
    L = 128
    R = P // L
    x_t = inputs.T.reshape(C, R, L)
    t2 = targets.reshape(R, L).astype(jnp.int32)
    out = pl.pallas_call(
        functools.partial(_lovasz_body, R=R, L=L, C=C),
        grid=(C,),
        in_specs=[
            pl.BlockSpec((1, R, L), lambda c: (c, 0, 0)),
            pl.BlockSpec((R, L), lambda c: (0, 0)),
        ],
        out_specs=pl.BlockSpec((1, 1), lambda c: (0, 0)),
        out_shape=jax.ShapeDtypeStruct((1, 1), jnp.float32),
        scratch_shapes=[pltpu.SMEM((2,), jnp.float32)],
    )(x_t, t2)
    return out[0, 0]
